# Initial kernel scaffold; baseline (speedup 1.0000x reference)
import jax
import jax.numpy as jnp
from jax.experimental import pallas as pl


def kernel(z, W):
    raise NotImplementedError("write your pallas kernel here")


# speedup vs baseline: 2.7159x; 2.7159x over previous
import functools

import jax
import jax.numpy as jnp
from jax.experimental import pallas as pl
from jax.experimental.pallas import tpu as pltpu

N_E = 1024
E_DIM = 256
N_B = 16
TOK = 1024
BETA = 0.25
N_TOTAL = N_B * TOK


def _vq_kernel(z_ref, z2_ref, w_ref, loss_ref, zq_ref, ppl_ref, enc_ref,
               idx_ref, counts_ref, loss_acc_ref):
    b = pl.program_id(0)
    z_b = z_ref[0]
    w = w_ref[...]

    s = jax.lax.dot_general(w.astype(jnp.bfloat16), z_b.astype(jnp.bfloat16),
                            (((1,), (0,)), ((), ())),
                            preferred_element_type=jnp.float32)
    w2 = jnp.sum(w * w, axis=1, keepdims=True)
    z2 = z2_ref[0]
    d = (z2 + w2) - 2.0 * s

    mins = jnp.min(d, axis=0, keepdims=True)
    cidx = jax.lax.broadcasted_iota(jnp.int32, (N_E, TOK), 0)
    idx = jnp.min(jnp.where(d == mins, cidx, N_E), axis=0)
    idx_ref[0, 0, :] = idx

    row_iota = jax.lax.broadcasted_iota(jnp.int32, (TOK, TOK), 0)
    col_iota = jax.lax.broadcasted_iota(jnp.int32, (TOK, TOK), 1)
    eye = jnp.where(row_iota == col_iota, 1.0, 0.0).astype(jnp.float32)
    idx_col = jax.lax.dot_general(
        eye, idx.astype(jnp.float32).reshape(1, TOK),
        (((1,), (1,)), ((), ())),
        preferred_element_type=jnp.float32,
        precision=jax.lax.Precision.HIGHEST)

    code_iota = jax.lax.broadcasted_iota(jnp.int32, (TOK, N_E), 1)
    onehot = jnp.where(idx_col.astype(jnp.int32) == code_iota,
                       1.0, 0.0).astype(jnp.float32)
    enc_ref[...] = onehot

    zq = jax.lax.dot_general(w, onehot, (((0,), (1,)), ((), ())),
                             preferred_element_type=jnp.float32)
    zq_ref[0] = z_b + (zq - z_b)

    @pl.when(b == 0)
    def _init():
        counts_ref[...] = jnp.zeros_like(counts_ref)
        loss_acc_ref[...] = jnp.zeros_like(loss_acc_ref)

    counts_ref[...] += jnp.sum(onehot, axis=0, keepdims=True)
    loss_acc_ref[...] += jnp.sum((zq - z_b) ** 2).reshape(1, 1)

    @pl.when(b == N_B - 1)
    def _finalize():
        m = loss_acc_ref[...] / jnp.float32(N_TOTAL * E_DIM)
        loss_ref[...] = m + BETA * m
        e_mean = counts_ref[...] * (1.0 / N_TOTAL)
        ppl_ref[...] = jnp.exp(
            -jnp.sum(e_mean * jnp.log(e_mean + 1e-10))).reshape(1, 1)


@jax.jit
def kernel(z, W):
    z_r = z.reshape(N_B, E_DIM, TOK)
    zp = jnp.transpose(z, (0, 2, 3, 1)).reshape(-1, E_DIM)
    z2 = jnp.sum(zp ** 2, axis=1).reshape(N_B, 1, TOK)
    loss, zq, ppl, enc, idxs = pl.pallas_call(
        _vq_kernel,
        grid=(N_B,),
        in_specs=[
            pl.BlockSpec((1, E_DIM, TOK), lambda b: (b, 0, 0)),
            pl.BlockSpec((1, 1, TOK), lambda b: (b, 0, 0)),
            pl.BlockSpec((N_E, E_DIM), lambda b: (0, 0)),
        ],
        out_specs=[
            pl.BlockSpec((1, 1), lambda b: (0, 0)),
            pl.BlockSpec((1, E_DIM, TOK), lambda b: (b, 0, 0)),
            pl.BlockSpec((1, 1), lambda b: (0, 0)),
            pl.BlockSpec((TOK, N_E), lambda b: (b, 0)),
            pl.BlockSpec((1, 1, TOK), lambda b: (b, 0, 0)),
        ],
        out_shape=[
            jax.ShapeDtypeStruct((1, 1), jnp.float32),
            jax.ShapeDtypeStruct((N_B, E_DIM, TOK), jnp.float32),
            jax.ShapeDtypeStruct((1, 1), jnp.float32),
            jax.ShapeDtypeStruct((N_B * TOK, N_E), jnp.float32),
            jax.ShapeDtypeStruct((N_B, 1, TOK), jnp.int32),
        ],
        scratch_shapes=[
            pltpu.VMEM((1, N_E), jnp.float32),
            pltpu.VMEM((1, 1), jnp.float32),
        ],
    )(z_r, z2, W)
    return (loss.reshape(()),
            zq.reshape(N_B, E_DIM, 32, 32),
            ppl.reshape(()),
            enc,
            idxs.reshape(N_B * TOK, 1))


# speedup vs baseline: 2.7814x; 1.0241x over previous
import jax
import jax.numpy as jnp
from jax.experimental import pallas as pl
from jax.experimental.pallas import tpu as pltpu

N_E = 1024
E_DIM = 256
N_B = 16
TOK = 1024
BETA = 0.25
N_TOTAL = N_B * TOK


def _vq_kernel(z_ref, z2_ref, w_ref, loss_ref, zq_ref, ppl_ref, enc_ref,
               idx_ref, counts_ref, loss_acc_ref, w2_ref, eye_ref):
    b = pl.program_id(0)
    z_b = z_ref[0]
    w = w_ref[...]

    @pl.when(b == 0)
    def _init():
        w2_ref[...] = jnp.sum(w * w, axis=1, keepdims=True)
        ri = jax.lax.broadcasted_iota(jnp.int32, (TOK, TOK), 0)
        ci = jax.lax.broadcasted_iota(jnp.int32, (TOK, TOK), 1)
        eye_ref[...] = jnp.where(ri == ci, 1.0, 0.0).astype(jnp.float32)
        counts_ref[...] = jnp.zeros_like(counts_ref)
        loss_acc_ref[...] = jnp.zeros_like(loss_acc_ref)

    s = jax.lax.dot_general(w.astype(jnp.bfloat16), z_b.astype(jnp.bfloat16),
                            (((1,), (0,)), ((), ())),
                            preferred_element_type=jnp.float32)
    d = (z2_ref[0] + w2_ref[...]) - 2.0 * s

    mins = jnp.min(d, axis=0, keepdims=True)
    cidx = jax.lax.broadcasted_iota(jnp.int32, (N_E, TOK), 0)
    idx = jnp.min(jnp.where(d == mins, cidx, N_E), axis=0)
    idx_ref[0, 0, :] = idx

    idx_col = jax.lax.dot_general(
        eye_ref[...], idx.astype(jnp.float32).reshape(1, TOK),
        (((1,), (1,)), ((), ())),
        preferred_element_type=jnp.float32,
        precision=jax.lax.Precision.HIGHEST)

    code_iota = jax.lax.broadcasted_iota(jnp.int32, (TOK, N_E), 1)
    hit = idx_col.astype(jnp.int32) == code_iota
    enc_ref[...] = jnp.where(hit, 1.0, 0.0).astype(jnp.float32)
    onehot_bf = jnp.where(hit, 1.0, 0.0).astype(jnp.bfloat16)

    zq = jax.lax.dot_general(w.astype(jnp.bfloat16), onehot_bf,
                             (((0,), (1,)), ((), ())),
                             preferred_element_type=jnp.float32)
    r = zq - z_b
    zq_ref[0] = z_b + r

    ones_row = jnp.full((1, TOK), 1.0, dtype=jnp.bfloat16)
    counts_ref[...] += jax.lax.dot_general(
        ones_row, onehot_bf, (((1,), (0,)), ((), ())),
        preferred_element_type=jnp.float32)
    loss_acc_ref[...] += jnp.sum(r * r, axis=0, keepdims=True)

    @pl.when(b == N_B - 1)
    def _finalize():
        m = jnp.sum(loss_acc_ref[...]) / jnp.float32(N_TOTAL * E_DIM)
        loss_ref[...] = (m + BETA * m).reshape(1, 1)
        e_mean = counts_ref[...] * (1.0 / N_TOTAL)
        ppl_ref[...] = jnp.exp(
            -jnp.sum(e_mean * jnp.log(e_mean + 1e-10))).reshape(1, 1)


@jax.jit
def kernel(z, W):
    z_r = z.reshape(N_B, E_DIM, TOK)
    zp = jnp.transpose(z, (0, 2, 3, 1)).reshape(-1, E_DIM)
    z2 = jnp.sum(zp ** 2, axis=1).reshape(N_B, 1, TOK)
    loss, zq, ppl, enc, idxs = pl.pallas_call(
        _vq_kernel,
        grid=(N_B,),
        in_specs=[
            pl.BlockSpec((1, E_DIM, TOK), lambda b: (b, 0, 0)),
            pl.BlockSpec((1, 1, TOK), lambda b: (b, 0, 0)),
            pl.BlockSpec((N_E, E_DIM), lambda b: (0, 0)),
        ],
        out_specs=[
            pl.BlockSpec((1, 1), lambda b: (0, 0)),
            pl.BlockSpec((1, E_DIM, TOK), lambda b: (b, 0, 0)),
            pl.BlockSpec((1, 1), lambda b: (0, 0)),
            pl.BlockSpec((TOK, N_E), lambda b: (b, 0)),
            pl.BlockSpec((1, 1, TOK), lambda b: (b, 0, 0)),
        ],
        out_shape=[
            jax.ShapeDtypeStruct((1, 1), jnp.float32),
            jax.ShapeDtypeStruct((N_B, E_DIM, TOK), jnp.float32),
            jax.ShapeDtypeStruct((1, 1), jnp.float32),
            jax.ShapeDtypeStruct((N_B * TOK, N_E), jnp.float32),
            jax.ShapeDtypeStruct((N_B, 1, TOK), jnp.int32),
        ],
        scratch_shapes=[
            pltpu.VMEM((1, N_E), jnp.float32),
            pltpu.VMEM((1, TOK), jnp.float32),
            pltpu.VMEM((N_E, 1), jnp.float32),
            pltpu.VMEM((TOK, TOK), jnp.float32),
        ],
    )(z_r, z2, W)
    return (loss.reshape(()),
            zq.reshape(N_B, E_DIM, 32, 32),
            ppl.reshape(()),
            enc,
            idxs.reshape(N_B * TOK, 1))


# speedup vs baseline: 2.7958x; 1.0052x over previous
import jax
import jax.numpy as jnp
from jax.experimental import pallas as pl
from jax.experimental.pallas import tpu as pltpu

N_E = 1024
E_DIM = 256
N_B = 16
TOK = 1024
BETA = 0.25
N_TOTAL = N_B * TOK


def _vq_kernel(z_ref, z2_ref, w_ref, loss_ref, zq_ref, ppl_ref, enc_ref,
               idx_ref, counts_ref, loss_acc_ref, w2_ref, eye_ref):
    b = pl.program_id(0)
    z_b = z_ref[0]
    w = w_ref[...]

    @pl.when(b == 0)
    def _init():
        w2_ref[...] = jnp.sum(w * w, axis=1, keepdims=True)
        ri = jax.lax.broadcasted_iota(jnp.int32, (TOK, TOK), 0)
        ci = jax.lax.broadcasted_iota(jnp.int32, (TOK, TOK), 1)
        eye_ref[...] = jnp.where(ri == ci, 1.0, 0.0).astype(jnp.float32)
        counts_ref[...] = jnp.zeros_like(counts_ref)
        loss_acc_ref[...] = jnp.zeros_like(loss_acc_ref)

    s = jax.lax.dot_general(w.astype(jnp.bfloat16), z_b.astype(jnp.bfloat16),
                            (((1,), (0,)), ((), ())),
                            preferred_element_type=jnp.float32)
    d = (z2_ref[0] + w2_ref[...]) - 2.0 * s

    mins = jnp.min(d, axis=0, keepdims=True)
    cidx = jax.lax.broadcasted_iota(jnp.int32, (N_E, TOK), 0)
    idx = jnp.min(jnp.where(d == mins, cidx, N_E), axis=0)
    idx_ref[0, 0, :] = idx

    idx_col = jax.lax.dot_general(
        eye_ref[...], idx.astype(jnp.float32).reshape(1, TOK),
        (((1,), (1,)), ((), ())),
        preferred_element_type=jnp.float32,
        precision=jax.lax.Precision.HIGHEST)

    code_iota = jax.lax.broadcasted_iota(jnp.int32, (TOK, N_E), 1)
    hit = idx_col.astype(jnp.int32) == code_iota
    enc_ref[...] = jnp.where(hit, 1.0, 0.0).astype(jnp.float32)
    onehot_bf = jnp.where(hit, 1.0, 0.0).astype(jnp.bfloat16)

    zq = jax.lax.dot_general(w.astype(jnp.bfloat16), onehot_bf,
                             (((0,), (1,)), ((), ())),
                             preferred_element_type=jnp.float32)
    r = zq - z_b
    zq_ref[0] = z_b + r

    ones_row = jnp.full((1, TOK), 1.0, dtype=jnp.bfloat16)
    counts_ref[...] += jax.lax.dot_general(
        ones_row, onehot_bf, (((1,), (0,)), ((), ())),
        preferred_element_type=jnp.float32)
    loss_acc_ref[...] += jnp.sum(r * r, axis=0, keepdims=True)

    @pl.when(b == N_B - 1)
    def _finalize():
        m = jnp.sum(loss_acc_ref[...]) / jnp.float32(N_TOTAL * E_DIM)
        loss_ref[...] = (m + BETA * m).reshape(1, 1)
        e_mean = counts_ref[...] * (1.0 / N_TOTAL)
        ppl_ref[...] = jnp.exp(
            -jnp.sum(e_mean * jnp.log(e_mean + 1e-10))).reshape(1, 1)


@jax.jit
def kernel(z, W):
    z_r = z.reshape(N_B, E_DIM, TOK)
    z2 = jnp.sum(z * z, axis=1).reshape(N_B, 1, TOK)
    loss, zq, ppl, enc, idxs = pl.pallas_call(
        _vq_kernel,
        grid=(N_B,),
        in_specs=[
            pl.BlockSpec((1, E_DIM, TOK), lambda b: (b, 0, 0)),
            pl.BlockSpec((1, 1, TOK), lambda b: (b, 0, 0)),
            pl.BlockSpec((N_E, E_DIM), lambda b: (0, 0)),
        ],
        out_specs=[
            pl.BlockSpec((1, 1), lambda b: (0, 0)),
            pl.BlockSpec((1, E_DIM, TOK), lambda b: (b, 0, 0)),
            pl.BlockSpec((1, 1), lambda b: (0, 0)),
            pl.BlockSpec((TOK, N_E), lambda b: (b, 0)),
            pl.BlockSpec((1, 1, TOK), lambda b: (b, 0, 0)),
        ],
        out_shape=[
            jax.ShapeDtypeStruct((1, 1), jnp.float32),
            jax.ShapeDtypeStruct((N_B, E_DIM, TOK), jnp.float32),
            jax.ShapeDtypeStruct((1, 1), jnp.float32),
            jax.ShapeDtypeStruct((N_B * TOK, N_E), jnp.float32),
            jax.ShapeDtypeStruct((N_B, 1, TOK), jnp.int32),
        ],
        scratch_shapes=[
            pltpu.VMEM((1, N_E), jnp.float32),
            pltpu.VMEM((1, TOK), jnp.float32),
            pltpu.VMEM((N_E, 1), jnp.float32),
            pltpu.VMEM((TOK, TOK), jnp.float32),
        ],
    )(z_r, z2, W)
    return (loss.reshape(()),
            zq.reshape(N_B, E_DIM, 32, 32),
            ppl.reshape(()),
            enc,
            idxs.reshape(N_B * TOK, 1))
